# baseline (speedup 1.0000x reference)
import functools

import jax
import jax.numpy as jnp
from jax.experimental import pallas as pl
from jax.experimental.pallas import tpu as pltpu


def _body(xr_ref, w0_ref, b0_ref, w1_ref, gamma_ref, beta_ref, y_ref,
          psum_ref, pssq_ref, scale_ref, shift_ref, *, n, nl, eps):
    s = pl.program_id(0)

    @pl.when(s == n)
    def _fold():
        m = jnp.float32(nl)
        mean = psum_ref[...] / m
        var = jnp.maximum(pssq_ref[...] / m - mean * mean, 0.0)
        scale = gamma_ref[...] * jax.lax.rsqrt(var + eps)
        scale_ref[...] = scale
        shift_ref[...] = beta_ref[...] - mean * scale

    def _stack(xb):
        h1 = jnp.maximum(
            jnp.dot(w0_ref[...], xb, preferred_element_type=jnp.float32)
            + b0_ref[...],
            0.0,
        )
        return jnp.dot(w1_ref[...], h1.astype(jnp.bfloat16),
                       preferred_element_type=jnp.float32)

    @pl.when(s < n)
    def _stats():
        z = _stack(xr_ref[s])
        h2 = jnp.maximum(z, 0.0)
        sm = jnp.sum(h2, axis=-1, keepdims=True)
        sq = jnp.sum(h2 * h2, axis=-1, keepdims=True)

        @pl.when(s == 0)
        def _init():
            psum_ref[...] = sm
            pssq_ref[...] = sq

        @pl.when(s != 0)
        def _acc():
            psum_ref[...] += sm
            pssq_ref[...] += sq

    @pl.when(s >= n)
    def _norm():
        z = _stack(xr_ref[s - n])
        y_ref[...] = jnp.maximum(z, 0.0) * scale_ref[...] + shift_ref[...]


def kernel(x, w0, b0, w1, b1, gamma, beta, eps=1e-5):
    N, C_in, L = x.shape
    C_mid = w0.shape[0]
    C_out = w1.shape[0]

    CM = ((C_mid + 1 + 7) // 8) * 8
    w0a = jnp.zeros((CM, C_in), jnp.float32).at[:C_mid].set(w0).astype(jnp.bfloat16)
    b0a = (
        jnp.zeros((CM, 1), jnp.float32)
        .at[:C_mid].set(b0)
        .at[C_mid, 0].set(1.0)
    )
    w1a = (
        jnp.zeros((C_out, CM), jnp.float32)
        .at[:, :C_mid].set(w1)
        .at[:, C_mid].set(b1[:, 0])
        .astype(jnp.bfloat16)
    )

    body = functools.partial(_body, n=N, nl=N * L, eps=eps)

    y = pl.pallas_call(
        body,
        out_shape=jax.ShapeDtypeStruct((N, C_out, L), x.dtype),
        grid=(2 * N,),
        in_specs=[
            pl.BlockSpec((N, C_in, L), lambda s: (0, 0, 0)),
            pl.BlockSpec((CM, C_in), lambda s: (0, 0)),
            pl.BlockSpec((CM, 1), lambda s: (0, 0)),
            pl.BlockSpec((C_out, CM), lambda s: (0, 0)),
            pl.BlockSpec((C_out, 1), lambda s: (0, 0)),
            pl.BlockSpec((C_out, 1), lambda s: (0, 0)),
        ],
        out_specs=pl.BlockSpec((None, C_out, L),
                               lambda s: (jnp.maximum(s - N, 0), 0, 0)),
        scratch_shapes=[
            pltpu.VMEM((C_out, 1), jnp.float32),
            pltpu.VMEM((C_out, 1), jnp.float32),
            pltpu.VMEM((C_out, 1), jnp.float32),
            pltpu.VMEM((C_out, 1), jnp.float32),
        ],
        compiler_params=pltpu.CompilerParams(
            dimension_semantics=("arbitrary",),
            vmem_limit_bytes=60 * 1024 * 1024,
        ),
    )(x.astype(jnp.bfloat16), w0a, b0a, w1a, gamma.astype(jnp.float32), beta.astype(jnp.float32))
    return y


# speedup vs baseline: 1.0325x; 1.0325x over previous
import functools

import jax
import jax.numpy as jnp
from jax.experimental import pallas as pl
from jax.experimental.pallas import tpu as pltpu


def _fused_body(x_ref, w0_ref, b0_ref, w1_ref, gamma_ref, beta_ref, y_ref,
                psum_ref, pssq_ref, scale_ref, shift_ref, *, nl, eps):
    p = pl.program_id(0)
    n = pl.program_id(1)

    @pl.when((p == 1) & (n == 0))
    def _fold():
        m = jnp.float32(nl)
        mean = psum_ref[...] / m
        var = jnp.maximum(pssq_ref[...] / m - mean * mean, 0.0)
        scale = gamma_ref[...] * jax.lax.rsqrt(var + eps)
        scale_ref[...] = scale
        shift_ref[...] = beta_ref[...] - mean * scale

    def _stack():
        xb = x_ref[n].astype(jnp.bfloat16)
        h1 = jnp.maximum(
            jnp.dot(w0_ref[...], xb, preferred_element_type=jnp.float32)
            + b0_ref[...],
            0.0,
        )
        return jnp.maximum(
            jnp.dot(w1_ref[...], h1.astype(jnp.bfloat16),
                    preferred_element_type=jnp.float32),
            0.0,
        )

    @pl.when(p == 0)
    def _stats():
        h2 = _stack()
        s = jnp.sum(h2, axis=-1, keepdims=True)
        q = jnp.sum(h2 * h2, axis=-1, keepdims=True)

        @pl.when(n == 0)
        def _init():
            psum_ref[...] = s
            pssq_ref[...] = q

        @pl.when(n != 0)
        def _acc():
            psum_ref[...] += s
            pssq_ref[...] += q

    @pl.when(p == 1)
    def _norm():
        h2 = _stack()
        y_ref[...] = h2 * scale_ref[...] + shift_ref[...]


def kernel(x, w0, b0, w1, b1, gamma, beta, eps=1e-5):
    N, C_in, L = x.shape
    C_mid = w0.shape[0]
    C_out = w1.shape[0]

    CM = ((C_mid + 1 + 7) // 8) * 8
    w0a = jnp.zeros((CM, C_in), jnp.float32).at[:C_mid].set(w0).astype(jnp.bfloat16)
    b0a = (
        jnp.zeros((CM, 1), jnp.float32)
        .at[:C_mid].set(b0)
        .at[C_mid, 0].set(1.0)
    )
    w1a = (
        jnp.zeros((C_out, CM), jnp.float32)
        .at[:, :C_mid].set(w1)
        .at[:, C_mid].set(b1[:, 0])
        .astype(jnp.bfloat16)
    )

    body = functools.partial(_fused_body, nl=N * L, eps=eps)

    y = pl.pallas_call(
        body,
        out_shape=jax.ShapeDtypeStruct((N, C_out, L), x.dtype),
        grid=(2, N),
        in_specs=[
            pl.BlockSpec((N, C_in, L), lambda p, n: (0, 0, 0)),
            pl.BlockSpec((CM, C_in), lambda p, n: (0, 0)),
            pl.BlockSpec((CM, 1), lambda p, n: (0, 0)),
            pl.BlockSpec((C_out, CM), lambda p, n: (0, 0)),
            pl.BlockSpec((C_out, 1), lambda p, n: (0, 0)),
            pl.BlockSpec((C_out, 1), lambda p, n: (0, 0)),
        ],
        out_specs=pl.BlockSpec((None, C_out, L), lambda p, n: (p * n, 0, 0)),
        scratch_shapes=[
            pltpu.VMEM((C_out, 1), jnp.float32),
            pltpu.VMEM((C_out, 1), jnp.float32),
            pltpu.VMEM((C_out, 1), jnp.float32),
            pltpu.VMEM((C_out, 1), jnp.float32),
        ],
        compiler_params=pltpu.CompilerParams(
            dimension_semantics=("arbitrary", "arbitrary"),
            vmem_limit_bytes=60 * 1024 * 1024,
        ),
    )(x, w0a, b0a, w1a, gamma.astype(jnp.float32), beta.astype(jnp.float32))
    return y


# speedup vs baseline: 1.0409x; 1.0082x over previous
import functools

import jax
import jax.numpy as jnp
from jax.experimental import pallas as pl
from jax.experimental.pallas import tpu as pltpu


def _fused_body(x_ref, w0_ref, b0_ref, w1_ref, w0f_ref, w1f_ref,
                gamma_ref, beta_ref, y_ref,
                psum_ref, pssq_ref, scale_ref, shift_ref, *, nl, eps):
    p = pl.program_id(0)
    n = pl.program_id(1)

    @pl.when((p == 1) & (n == 0))
    def _fold():
        m = jnp.float32(nl)
        mean = psum_ref[...] / m
        var = jnp.maximum(pssq_ref[...] / m - mean * mean, 0.0)
        scale = gamma_ref[...] * jax.lax.rsqrt(var + eps)
        scale_ref[...] = scale
        shift_ref[...] = beta_ref[...] - mean * scale

    def _stack():
        xb = x_ref[n].astype(jnp.bfloat16)
        h1 = jnp.maximum(
            jnp.dot(w0_ref[...], xb, preferred_element_type=jnp.float32)
            + b0_ref[...],
            0.0,
        )
        return jnp.maximum(
            jnp.dot(w1_ref[...], h1.astype(jnp.bfloat16),
                    preferred_element_type=jnp.float32),
            0.0,
        )

    @pl.when(p == 0)
    def _stats():
        h2 = _stack()
        s = jnp.sum(h2, axis=-1, keepdims=True)
        q = jnp.sum(h2 * h2, axis=-1, keepdims=True)

        @pl.when(n == 0)
        def _init():
            psum_ref[...] = s
            pssq_ref[...] = q

        @pl.when(n != 0)
        def _acc():
            psum_ref[...] += s
            pssq_ref[...] += q

    @pl.when(p == 1)
    def _norm():
        h1 = jnp.maximum(
            jnp.dot(w0f_ref[...], x_ref[n], preferred_element_type=jnp.float32)
            + b0_ref[...],
            0.0,
        )
        h2 = jnp.maximum(
            jnp.dot(w1f_ref[...], h1, preferred_element_type=jnp.float32), 0.0)
        y_ref[...] = h2 * scale_ref[...] + shift_ref[...]


def kernel(x, w0, b0, w1, b1, gamma, beta, eps=1e-5):
    N, C_in, L = x.shape
    C_mid = w0.shape[0]
    C_out = w1.shape[0]

    CM = ((C_mid + 1 + 7) // 8) * 8
    w0a = jnp.zeros((CM, C_in), jnp.float32).at[:C_mid].set(w0).astype(jnp.bfloat16)
    b0a = (
        jnp.zeros((CM, 1), jnp.float32)
        .at[:C_mid].set(b0)
        .at[C_mid, 0].set(1.0)
    )
    w1a = (
        jnp.zeros((C_out, CM), jnp.float32)
        .at[:, :C_mid].set(w1)
        .at[:, C_mid].set(b1[:, 0])
        .astype(jnp.bfloat16)
    )

    body = functools.partial(_fused_body, nl=N * L, eps=eps)

    y = pl.pallas_call(
        body,
        out_shape=jax.ShapeDtypeStruct((N, C_out, L), x.dtype),
        grid=(2, N),
        in_specs=[
            pl.BlockSpec((N, C_in, L), lambda p, n: (0, 0, 0)),
            pl.BlockSpec((CM, C_in), lambda p, n: (0, 0)),
            pl.BlockSpec((CM, 1), lambda p, n: (0, 0)),
            pl.BlockSpec((C_out, CM), lambda p, n: (0, 0)),
            pl.BlockSpec((CM, C_in), lambda p, n: (0, 0)),
            pl.BlockSpec((C_out, CM), lambda p, n: (0, 0)),
            pl.BlockSpec((C_out, 1), lambda p, n: (0, 0)),
            pl.BlockSpec((C_out, 1), lambda p, n: (0, 0)),
        ],
        out_specs=pl.BlockSpec((None, C_out, L), lambda p, n: (p * n, 0, 0)),
        scratch_shapes=[
            pltpu.VMEM((C_out, 1), jnp.float32),
            pltpu.VMEM((C_out, 1), jnp.float32),
            pltpu.VMEM((C_out, 1), jnp.float32),
            pltpu.VMEM((C_out, 1), jnp.float32),
        ],
        compiler_params=pltpu.CompilerParams(
            dimension_semantics=("arbitrary", "arbitrary"),
            vmem_limit_bytes=60 * 1024 * 1024,
        ),
    )(x, w0a, b0a, w1a, w0a.astype(jnp.float32), w1a.astype(jnp.float32),
      gamma.astype(jnp.float32), beta.astype(jnp.float32))
    return y


# speedup vs baseline: 1.0619x; 1.0202x over previous
import functools

import jax
import jax.numpy as jnp
from jax.experimental import pallas as pl
from jax.experimental.pallas import tpu as pltpu


def _fused_body(x_ref, w0_ref, b0_ref, w1_ref, w0f_ref, w1f_ref,
                gamma_ref, beta_ref, y_ref,
                psum_ref, pssq_ref, scale_ref, shift_ref, *, nl, eps):
    p = pl.program_id(0)
    n = pl.program_id(1)

    @pl.when((p == 1) & (n == 0))
    def _fold():
        m = jnp.float32(nl)
        mean = psum_ref[...] / m
        var = jnp.maximum(pssq_ref[...] / m - mean * mean, 0.0)
        scale = gamma_ref[...] * jax.lax.rsqrt(var + eps)
        scale_ref[...] = scale
        shift_ref[...] = beta_ref[...] - mean * scale

    def _stack():
        xb = x_ref[n].astype(jnp.bfloat16)
        h1 = jnp.maximum(
            jnp.dot(w0_ref[...], xb, preferred_element_type=jnp.float32)
            + b0_ref[...],
            0.0,
        )
        return jnp.maximum(
            jnp.dot(w1_ref[...], h1.astype(jnp.bfloat16),
                    preferred_element_type=jnp.float32),
            0.0,
        )

    @pl.when(p == 0)
    def _stats():
        h1 = jnp.maximum(
            jnp.dot(w0f_ref[...], x_ref[n], preferred_element_type=jnp.float32)
            + b0_ref[...],
            0.0,
        )
        h2 = jnp.maximum(
            jnp.dot(w1f_ref[...], h1, preferred_element_type=jnp.float32), 0.0)
        s = jnp.sum(h2, axis=-1, keepdims=True)
        q = jnp.sum(h2 * h2, axis=-1, keepdims=True)

        @pl.when(n == 0)
        def _init():
            psum_ref[...] = s
            pssq_ref[...] = q

        @pl.when(n != 0)
        def _acc():
            psum_ref[...] += s
            pssq_ref[...] += q

    @pl.when(p == 1)
    def _norm():
        h1 = jnp.maximum(
            jnp.dot(w0f_ref[...], x_ref[n], preferred_element_type=jnp.float32)
            + b0_ref[...],
            0.0,
        )
        h2 = jnp.maximum(
            jnp.dot(w1f_ref[...], h1, preferred_element_type=jnp.float32), 0.0)
        y_ref[...] = h2 * scale_ref[...] + shift_ref[...]


def kernel(x, w0, b0, w1, b1, gamma, beta, eps=1e-5):
    N, C_in, L = x.shape
    C_mid = w0.shape[0]
    C_out = w1.shape[0]

    CM = ((C_mid + 1 + 7) // 8) * 8
    w0a = jnp.zeros((CM, C_in), jnp.float32).at[:C_mid].set(w0).astype(jnp.bfloat16)
    b0a = (
        jnp.zeros((CM, 1), jnp.float32)
        .at[:C_mid].set(b0)
        .at[C_mid, 0].set(1.0)
    )
    w1a = (
        jnp.zeros((C_out, CM), jnp.float32)
        .at[:, :C_mid].set(w1)
        .at[:, C_mid].set(b1[:, 0])
        .astype(jnp.bfloat16)
    )

    body = functools.partial(_fused_body, nl=N * L, eps=eps)

    y = pl.pallas_call(
        body,
        out_shape=jax.ShapeDtypeStruct((N, C_out, L), x.dtype),
        grid=(2, N),
        in_specs=[
            pl.BlockSpec((N, C_in, L), lambda p, n: (0, 0, 0)),
            pl.BlockSpec((CM, C_in), lambda p, n: (0, 0)),
            pl.BlockSpec((CM, 1), lambda p, n: (0, 0)),
            pl.BlockSpec((C_out, CM), lambda p, n: (0, 0)),
            pl.BlockSpec((CM, C_in), lambda p, n: (0, 0)),
            pl.BlockSpec((C_out, CM), lambda p, n: (0, 0)),
            pl.BlockSpec((C_out, 1), lambda p, n: (0, 0)),
            pl.BlockSpec((C_out, 1), lambda p, n: (0, 0)),
        ],
        out_specs=pl.BlockSpec((None, C_out, L), lambda p, n: (p * n, 0, 0)),
        scratch_shapes=[
            pltpu.VMEM((C_out, 1), jnp.float32),
            pltpu.VMEM((C_out, 1), jnp.float32),
            pltpu.VMEM((C_out, 1), jnp.float32),
            pltpu.VMEM((C_out, 1), jnp.float32),
        ],
        compiler_params=pltpu.CompilerParams(
            dimension_semantics=("arbitrary", "arbitrary"),
            vmem_limit_bytes=60 * 1024 * 1024,
        ),
    )(x, w0a, b0a, w1a, w0a.astype(jnp.float32), w1a.astype(jnp.float32),
      gamma.astype(jnp.float32), beta.astype(jnp.float32))
    return y


# speedup vs baseline: 1.0670x; 1.0048x over previous
import functools

import jax
import jax.numpy as jnp
from jax.experimental import pallas as pl
from jax.experimental.pallas import tpu as pltpu


def _fused_body(x_ref, w0f_ref, b0_ref, w1f_ref,
                gamma_ref, beta_ref, y_ref,
                psum_ref, pssq_ref, scale_ref, shift_ref, *, nl, eps):
    p = pl.program_id(0)
    n = pl.program_id(1)

    @pl.when((p == 1) & (n == 0))
    def _fold():
        m = jnp.float32(nl)
        mean = psum_ref[...] / m
        var = jnp.maximum(pssq_ref[...] / m - mean * mean, 0.0)
        scale = gamma_ref[...] * jax.lax.rsqrt(var + eps)
        scale_ref[...] = scale
        shift_ref[...] = beta_ref[...] - mean * scale

    @pl.when(p == 0)
    def _stats():
        h1 = jnp.maximum(
            jnp.dot(w0f_ref[...], x_ref[n], preferred_element_type=jnp.float32)
            + b0_ref[...],
            0.0,
        )
        h2 = jnp.maximum(
            jnp.dot(w1f_ref[...], h1, preferred_element_type=jnp.float32), 0.0)
        s = jnp.sum(h2, axis=-1, keepdims=True)
        q = jnp.sum(h2 * h2, axis=-1, keepdims=True)

        @pl.when(n == 0)
        def _init():
            psum_ref[...] = s
            pssq_ref[...] = q

        @pl.when(n != 0)
        def _acc():
            psum_ref[...] += s
            pssq_ref[...] += q

    @pl.when(p == 1)
    def _norm():
        h1 = jnp.maximum(
            jnp.dot(w0f_ref[...], x_ref[n], preferred_element_type=jnp.float32)
            + b0_ref[...],
            0.0,
        )
        h2 = jnp.maximum(
            jnp.dot(w1f_ref[...], h1, preferred_element_type=jnp.float32), 0.0)
        y_ref[...] = h2 * scale_ref[...] + shift_ref[...]


def kernel(x, w0, b0, w1, b1, gamma, beta, eps=1e-5):
    N, C_in, L = x.shape
    C_mid = w0.shape[0]
    C_out = w1.shape[0]

    CM = ((C_mid + 1 + 7) // 8) * 8
    w0a = jnp.zeros((CM, C_in), jnp.float32).at[:C_mid].set(w0)
    b0a = (
        jnp.zeros((CM, 1), jnp.float32)
        .at[:C_mid].set(b0)
        .at[C_mid, 0].set(1.0)
    )
    w1a = (
        jnp.zeros((C_out, CM), jnp.float32)
        .at[:, :C_mid].set(w1)
        .at[:, C_mid].set(b1[:, 0])
    )

    body = functools.partial(_fused_body, nl=N * L, eps=eps)

    y = pl.pallas_call(
        body,
        out_shape=jax.ShapeDtypeStruct((N, C_out, L), x.dtype),
        grid=(2, N),
        in_specs=[
            pl.BlockSpec((N, C_in, L), lambda p, n: (0, 0, 0)),
            pl.BlockSpec((CM, C_in), lambda p, n: (0, 0)),
            pl.BlockSpec((CM, 1), lambda p, n: (0, 0)),
            pl.BlockSpec((C_out, CM), lambda p, n: (0, 0)),
            pl.BlockSpec((C_out, 1), lambda p, n: (0, 0)),
            pl.BlockSpec((C_out, 1), lambda p, n: (0, 0)),
        ],
        out_specs=pl.BlockSpec((None, C_out, L), lambda p, n: (p * n, 0, 0)),
        scratch_shapes=[
            pltpu.VMEM((C_out, 1), jnp.float32),
            pltpu.VMEM((C_out, 1), jnp.float32),
            pltpu.VMEM((C_out, 1), jnp.float32),
            pltpu.VMEM((C_out, 1), jnp.float32),
        ],
        compiler_params=pltpu.CompilerParams(
            dimension_semantics=("arbitrary", "arbitrary"),
            vmem_limit_bytes=60 * 1024 * 1024,
        ),
    )(x, w0a, b0a, w1a, gamma.astype(jnp.float32), beta.astype(jnp.float32))
    return y
